# baseline (speedup 1.0000x reference)
import functools

import jax
import jax.numpy as jnp
from jax import lax
from jax.experimental import pallas as pl
from jax.experimental.pallas import tpu as pltpu
from jax.experimental.pallas import tpu_sc as plsc


_CH = 128
_SLAB = 2048


def _sc_gather(hist_idx, hist_tab, user_id, user_tab, movie_id, item_tab):
    ntot = hist_idx.shape[0]
    b = user_id.shape[0]
    d = hist_tab.shape[1]
    pack = 128 // d
    info = plsc.get_sparse_core_info()
    nc, ns = info.num_cores, info.num_subcores
    nw = nc * ns
    assert ntot % (nw * _SLAB) == 0 and b % (nw * _CH) == 0
    rpw = ntot // nw
    nslab = rpw // _SLAB
    nch = _SLAB // _CH
    bpw = b // nw
    nch_b = bpw // _CH

    mesh = plsc.VectorSubcoreMesh(core_axis_name="c", subcore_axis_name="s")

    @functools.partial(
        pl.kernel,
        out_type=(
            jax.ShapeDtypeStruct((ntot, d), jnp.float32),
            jax.ShapeDtypeStruct((b, d), jnp.float32),
            jax.ShapeDtypeStruct((b, d), jnp.float32),
        ),
        mesh=mesh,
        compiler_params=pltpu.CompilerParams(use_tc_tiling_on_sc=False),
        scratch_types=[
            pltpu.VMEM((_SLAB,), jnp.int32),
            pltpu.VMEM((_SLAB, d), jnp.float32),
            pltpu.SemaphoreType.DMA,
        ],
    )
    def k(hidx, htab, uid, utab, mid, itab, hist_out, user_out, item_out,
          idx_v, rows_v, sem):
        wid = lax.axis_index("s") * nc + lax.axis_index("c")

        def gather_slab(idx_hbm, tab_hbm, off, n):
            pltpu.sync_copy(idx_hbm.at[pl.ds(off, n * _CH)],
                            idx_v.at[pl.ds(0, n * _CH)])
            cps = [
                pltpu.async_copy(tab_hbm.at[idx_v.at[pl.ds(c * _CH, _CH)]],
                                 rows_v.at[pl.ds(c * _CH, _CH)], sem)
                for c in range(n)
            ]
            for cp in cps:
                cp.wait()

        base = wid * rpw

        def body(s, carry):
            off = base + s * _SLAB
            gather_slab(hidx, htab, off, nch)
            pltpu.sync_copy(rows_v, hist_out.at[pl.ds(off, _SLAB)])
            return carry

        lax.fori_loop(0, nslab, body, 0)

        gather_slab(uid, utab, wid * bpw, nch_b)
        pltpu.sync_copy(rows_v.at[pl.ds(0, bpw)],
                        user_out.at[pl.ds(wid * bpw, bpw)])
        gather_slab(mid, itab, wid * bpw, nch_b)
        pltpu.sync_copy(rows_v.at[pl.ds(0, bpw)],
                        item_out.at[pl.ds(wid * bpw, bpw)])

    return k(hist_idx, hist_tab, user_id, user_tab, movie_id, item_tab)


def _tc_forward(hist_pack, user_e, item_e, dense, genres,
                w8h, w1t, b1, w2blk, e8, f128,
                bw1, bb1, bw2, bb2, gw, gb,
                tw1, tb1, tw2, tb2, tw3, tb3, l, bt):
    b, d = user_e.shape
    pack = 128 // d
    lp = l // pack
    nd = dense.shape[1]
    g = genres.shape[1]
    grid = (b // bt,)
    npk = bt * lp

    seg = jnp.kron(jnp.eye(bt, dtype=jnp.float32),
                   jnp.ones((1, lp), jnp.float32))
    segt = seg.T

    def body(hist_ref, ue_ref, ie_ref, de_ref, ge_ref,
             w8h_ref, w1t_ref, b1_ref, w2blk_ref, e8_ref, f128_ref,
             seg_ref, segt_ref,
             bw1_ref, bb1_ref, bw2_ref, bb2_ref, gw_ref, gb_ref,
             tw1_ref, tb1_ref, tw2_ref, tb2_ref, tw3_ref, tb3_ref,
             out_ref):
        hp = hist_ref[...]
        t = ie_ref[...]
        t8 = jnp.concatenate([t] * pack, axis=1)
        tx = jnp.dot(segt_ref[...], t8, preferred_element_type=jnp.float32)
        pp = hp * tx
        x = jnp.concatenate([hp, pp], axis=1)
        hk = jnp.dot(x, w8h_ref[...], preferred_element_type=jnp.float32)
        ct = jnp.dot(t, w1t_ref[...], preferred_element_type=jnp.float32)
        ct = ct + b1_ref[...]
        ct8 = jnp.concatenate([ct] * pack, axis=1)
        ctx = jnp.dot(segt_ref[...], ct8, preferred_element_type=jnp.float32)
        a = jnp.maximum(hk + ctx, 0)
        sp = jnp.dot(a, w2blk_ref[...], preferred_element_type=jnp.float32)
        spad = jnp.dot(jnp.maximum(ct8, 0), w2blk_ref[...],
                       preferred_element_type=jnp.float32)
        spt = sp.T
        spadx = jnp.dot(spad.T, seg_ref[...],
                        preferred_element_type=jnp.float32)
        sm = jnp.where(spt == spadx, -1e9, spt)
        et = jnp.exp(sm)
        cs = jnp.sum(et, axis=0, keepdims=True)
        zb = jnp.dot(cs, segt_ref[...], preferred_element_type=jnp.float32)
        rz = 1.0 / jnp.maximum(zb, 1e-30)
        rzx = jnp.dot(rz, seg_ref[...], preferred_element_type=jnp.float32)
        wp = (et * rzx).T
        wexp = jnp.dot(wp, e8_ref[...], preferred_element_type=jnp.float32)
        hw = hp * wexp
        whs = jnp.dot(seg_ref[...], hw,
                      preferred_element_type=jnp.float32)
        hist_e = jnp.dot(whs, f128_ref[...],
                         preferred_element_type=jnp.float32)

        d1 = jnp.maximum(jnp.dot(de_ref[...], bw1_ref[...],
                                 preferred_element_type=jnp.float32)
                         + bb1_ref[...], 0.0)
        dense_e = jnp.maximum(jnp.dot(d1, bw2_ref[...],
                                      preferred_element_type=jnp.float32)
                              + bb2_ref[...], 0.0)
        genre_e = jnp.maximum(jnp.dot(ge_ref[...], gw_ref[...],
                                      preferred_element_type=jnp.float32)
                              + gb_ref[...], 0.0)
        vecs = [ue_ref[...], t, hist_e, dense_e, genre_e]
        dots = []
        for i in range(5):
            for j in range(i + 1, 5):
                dots.append(jnp.sum(vecs[i] * vecs[j], axis=-1, keepdims=True))
        cat = jnp.concatenate(dots + vecs, axis=-1)
        x = jnp.maximum(jnp.dot(cat, tw1_ref[...],
                                preferred_element_type=jnp.float32)
                        + tb1_ref[...], 0.0)
        x = jnp.maximum(jnp.dot(x, tw2_ref[...],
                                preferred_element_type=jnp.float32)
                        + tb2_ref[...], 0.0)
        y = jnp.dot(x, tw3_ref[...], preferred_element_type=jnp.float32)
        out_ref[...] = y + tb3_ref[0, 0]

    row = lambda i: (i, 0)
    fixed = lambda i: (0, 0)
    consts = [w8h, w1t, b1, w2blk, e8, f128, seg, segt,
              bw1, bb1, bw2, bb2, gw, gb, tw1, tb1, tw2, tb2, tw3, tb3]
    return pl.pallas_call(
        body,
        grid=grid,
        in_specs=[
            pl.BlockSpec((npk, 128), row),
            pl.BlockSpec((bt, d), row),
            pl.BlockSpec((bt, d), row),
            pl.BlockSpec((bt, nd), row),
            pl.BlockSpec((bt, g), row),
        ] + [pl.BlockSpec(c.shape, fixed) for c in consts],
        out_specs=pl.BlockSpec((bt, 1), row),
        out_shape=jax.ShapeDtypeStruct((b, 1), jnp.float32),
    )(hist_pack, user_e, item_e, dense, genres, *consts)


def kernel(user_id, movie_id, dense, history, genres,
           user_table, item_table, hist_table,
           din_w1, din_b1, din_w2, din_b2,
           bot_w1, bot_b1, bot_w2, bot_b2,
           gen_w, gen_b,
           top_w1, top_b1, top_w2, top_b2, top_w3, top_b3):
    b, l = history.shape
    d = hist_table.shape[1]
    pack = 128 // d
    pad = hist_table.shape[0] - 1

    nchunk = 2
    bc = b // nchunk
    chunks = []
    for ci in range(nchunk):
        s = slice(ci * bc, (ci + 1) * bc)
        chunks.append(_sc_gather(
            history[s].reshape(bc * l), hist_table, user_id[s], user_table,
            movie_id[s], item_table))

    eye8 = jnp.eye(pack, dtype=jnp.float32)
    w8h = jnp.concatenate([jnp.kron(eye8, din_w1[:d]),
                           jnp.kron(eye8, din_w1[2 * d:])])
    w2blk = jnp.kron(eye8, din_w2)
    e8 = jnp.kron(eye8, jnp.ones((1, d), jnp.float32))
    f128 = jnp.kron(jnp.ones((pack, 1), jnp.float32),
                    jnp.eye(d, dtype=jnp.float32))

    outs = []
    for ci in range(nchunk):
        s = slice(ci * bc, (ci + 1) * bc)
        hist2, user_e, item_e = chunks[ci]
        out = _tc_forward(
            hist2.reshape(bc * l // pack, d * pack), user_e, item_e,
            dense[s], genres[s],
            w8h, din_w1[d:2 * d], din_b1.reshape(1, -1), w2blk, e8, f128,
            bot_w1, bot_b1.reshape(1, -1), bot_w2, bot_b2.reshape(1, -1),
            gen_w, gen_b.reshape(1, -1),
            top_w1, top_b1.reshape(1, -1), top_w2, top_b2.reshape(1, -1),
            top_w3, top_b3.reshape(1, 1), l, bt=128)
        outs.append(out[:, 0])
    return jnp.concatenate(outs)


# speedup vs baseline: 1.0007x; 1.0007x over previous
import functools

import jax
import jax.numpy as jnp
from jax import lax
from jax.experimental import pallas as pl
from jax.experimental.pallas import tpu as pltpu
from jax.experimental.pallas import tpu_sc as plsc


_CH = 128
_SLAB = 2048


def _sc_gather(hist_idx, hist_tab, user_id, user_tab, movie_id, item_tab):
    ntot = hist_idx.shape[0]
    b = user_id.shape[0]
    d = hist_tab.shape[1]
    pack = 128 // d
    info = plsc.get_sparse_core_info()
    nc, ns = info.num_cores, info.num_subcores
    nw = nc * ns
    assert ntot % (nw * _SLAB) == 0 and b % (nw * _CH) == 0
    rpw = ntot // nw
    nslab = rpw // _SLAB
    nch = _SLAB // _CH
    bpw = b // nw
    nch_b = bpw // _CH

    mesh = plsc.VectorSubcoreMesh(core_axis_name="c", subcore_axis_name="s")

    @functools.partial(
        pl.kernel,
        out_type=(
            jax.ShapeDtypeStruct((ntot, d), jnp.float32),
            jax.ShapeDtypeStruct((b, d), jnp.float32),
            jax.ShapeDtypeStruct((b, d), jnp.float32),
        ),
        mesh=mesh,
        compiler_params=pltpu.CompilerParams(use_tc_tiling_on_sc=False),
        scratch_types=[
            pltpu.VMEM((_SLAB,), jnp.int32),
            pltpu.VMEM((_SLAB, d), jnp.float32),
            pltpu.SemaphoreType.DMA,
        ],
    )
    def k(hidx, htab, uid, utab, mid, itab, hist_out, user_out, item_out,
          idx_v, rows_v, sem):
        wid = lax.axis_index("s") * nc + lax.axis_index("c")

        def gather_slab(idx_hbm, tab_hbm, off, n):
            pltpu.sync_copy(idx_hbm.at[pl.ds(off, n * _CH)],
                            idx_v.at[pl.ds(0, n * _CH)])
            cps = [
                pltpu.async_copy(tab_hbm.at[idx_v.at[pl.ds(c * _CH, _CH)]],
                                 rows_v.at[pl.ds(c * _CH, _CH)], sem)
                for c in range(n)
            ]
            for cp in cps:
                cp.wait()

        base = wid * rpw

        def body(s, carry):
            off = base + s * _SLAB
            gather_slab(hidx, htab, off, nch)
            pltpu.sync_copy(rows_v, hist_out.at[pl.ds(off, _SLAB)])
            return carry

        lax.fori_loop(0, nslab, body, 0)

        gather_slab(uid, utab, wid * bpw, nch_b)
        pltpu.sync_copy(rows_v.at[pl.ds(0, bpw)],
                        user_out.at[pl.ds(wid * bpw, bpw)])
        gather_slab(mid, itab, wid * bpw, nch_b)
        pltpu.sync_copy(rows_v.at[pl.ds(0, bpw)],
                        item_out.at[pl.ds(wid * bpw, bpw)])

    return k(hist_idx, hist_tab, user_id, user_tab, movie_id, item_tab)


def _tc_forward(hist_pack, user_e, item_e, dense, genres,
                w8h, w1t, b1, w2blk, e8, f128,
                bw1, bb1, bw2, bb2, gw, gb,
                tw1, tb1, tw2, tb2, tw3, tb3, l, bt):
    b, d = user_e.shape
    pack = 128 // d
    lp = l // pack
    nd = dense.shape[1]
    g = genres.shape[1]
    grid = (b // bt,)
    npk = bt * lp

    seg = jnp.kron(jnp.eye(bt, dtype=jnp.float32),
                   jnp.ones((1, lp), jnp.float32))
    segt = seg.T

    def body(hist_ref, ue_ref, ie_ref, de_ref, ge_ref,
             w8h_ref, w1t_ref, b1_ref, w2blk_ref, e8_ref, f128_ref,
             seg_ref, segt_ref,
             bw1_ref, bb1_ref, bw2_ref, bb2_ref, gw_ref, gb_ref,
             tw1_ref, tb1_ref, tw2_ref, tb2_ref, tw3_ref, tb3_ref,
             out_ref):
        hp = hist_ref[...]
        t = ie_ref[...]
        t8 = jnp.concatenate([t] * pack, axis=1)
        tx = jnp.dot(segt_ref[...], t8, preferred_element_type=jnp.float32)
        pp = hp * tx
        x = jnp.concatenate([hp, pp], axis=1)
        hk = jnp.dot(x, w8h_ref[...], preferred_element_type=jnp.float32)
        ct = jnp.dot(t, w1t_ref[...], preferred_element_type=jnp.float32)
        ct = ct + b1_ref[...]
        ct8 = jnp.concatenate([ct] * pack, axis=1)
        ctx = jnp.dot(segt_ref[...], ct8, preferred_element_type=jnp.float32)
        a = jnp.maximum(hk + ctx, 0)
        sp = jnp.dot(a, w2blk_ref[...], preferred_element_type=jnp.float32)
        spad = jnp.dot(jnp.maximum(ct8, 0), w2blk_ref[...],
                       preferred_element_type=jnp.float32)
        spt = sp.T
        spadx = jnp.dot(spad.T, seg_ref[...],
                        preferred_element_type=jnp.float32)
        sm = jnp.where(spt == spadx, -1e9, spt)
        et = jnp.exp(sm)
        cs = jnp.sum(et, axis=0, keepdims=True)
        zb = jnp.dot(cs, segt_ref[...], preferred_element_type=jnp.float32)
        rz = 1.0 / jnp.maximum(zb, 1e-30)
        rzx = jnp.dot(rz, seg_ref[...], preferred_element_type=jnp.float32)
        wp = (et * rzx).T
        wexp = jnp.dot(wp, e8_ref[...], preferred_element_type=jnp.float32)
        hw = hp * wexp
        whs = jnp.dot(seg_ref[...], hw,
                      preferred_element_type=jnp.float32)
        hist_e = jnp.dot(whs, f128_ref[...],
                         preferred_element_type=jnp.float32)

        d1 = jnp.maximum(jnp.dot(de_ref[...], bw1_ref[...],
                                 preferred_element_type=jnp.float32)
                         + bb1_ref[...], 0.0)
        dense_e = jnp.maximum(jnp.dot(d1, bw2_ref[...],
                                      preferred_element_type=jnp.float32)
                              + bb2_ref[...], 0.0)
        genre_e = jnp.maximum(jnp.dot(ge_ref[...], gw_ref[...],
                                      preferred_element_type=jnp.float32)
                              + gb_ref[...], 0.0)
        vecs = [ue_ref[...], t, hist_e, dense_e, genre_e]
        dots = []
        for i in range(5):
            for j in range(i + 1, 5):
                dots.append(jnp.sum(vecs[i] * vecs[j], axis=-1, keepdims=True))
        cat = jnp.concatenate(dots + vecs, axis=-1)
        x = jnp.maximum(jnp.dot(cat, tw1_ref[...],
                                preferred_element_type=jnp.float32)
                        + tb1_ref[...], 0.0)
        x = jnp.maximum(jnp.dot(x, tw2_ref[...],
                                preferred_element_type=jnp.float32)
                        + tb2_ref[...], 0.0)
        y = jnp.dot(x, tw3_ref[...], preferred_element_type=jnp.float32)
        out_ref[...] = y + tb3_ref[0, 0]

    row = lambda i: (i, 0)
    fixed = lambda i: (0, 0)
    consts = [w8h, w1t, b1, w2blk, e8, f128, seg, segt,
              bw1, bb1, bw2, bb2, gw, gb, tw1, tb1, tw2, tb2, tw3, tb3]
    return pl.pallas_call(
        body,
        grid=grid,
        in_specs=[
            pl.BlockSpec((npk, 128), row),
            pl.BlockSpec((bt, d), row),
            pl.BlockSpec((bt, d), row),
            pl.BlockSpec((bt, nd), row),
            pl.BlockSpec((bt, g), row),
        ] + [pl.BlockSpec(c.shape, fixed) for c in consts],
        out_specs=pl.BlockSpec((bt, 1), row),
        out_shape=jax.ShapeDtypeStruct((b, 1), jnp.float32),
    )(hist_pack, user_e, item_e, dense, genres, *consts)


def kernel(user_id, movie_id, dense, history, genres,
           user_table, item_table, hist_table,
           din_w1, din_b1, din_w2, din_b2,
           bot_w1, bot_b1, bot_w2, bot_b2,
           gen_w, gen_b,
           top_w1, top_b1, top_w2, top_b2, top_w3, top_b3):
    b, l = history.shape
    d = hist_table.shape[1]
    pack = 128 // d
    pad = hist_table.shape[0] - 1

    hist2, user_e, item_e = _sc_gather(
        history.reshape(b * l), hist_table, user_id, user_table,
        movie_id, item_table)
    hist_pack = hist2.reshape(b * l // pack, d * pack)

    eye8 = jnp.eye(pack, dtype=jnp.float32)
    w8h = jnp.concatenate([jnp.kron(eye8, din_w1[:d]),
                           jnp.kron(eye8, din_w1[2 * d:])])
    w2blk = jnp.kron(eye8, din_w2)
    e8 = jnp.kron(eye8, jnp.ones((1, d), jnp.float32))
    f128 = jnp.kron(jnp.ones((pack, 1), jnp.float32),
                    jnp.eye(d, dtype=jnp.float32))

    out = _tc_forward(
        hist_pack, user_e, item_e, dense, genres,
        w8h, din_w1[d:2 * d], din_b1.reshape(1, -1), w2blk, e8, f128,
        bot_w1, bot_b1.reshape(1, -1), bot_w2, bot_b2.reshape(1, -1),
        gen_w, gen_b.reshape(1, -1),
        top_w1, top_b1.reshape(1, -1), top_w2, top_b2.reshape(1, -1),
        top_w3, top_b3.reshape(1, 1), l, bt=128)
    return out[:, 0]
